# baseline (speedup 1.0000x reference)
import functools

import jax
import jax.numpy as jnp
from jax import lax
from jax.experimental import pallas as pl
from jax.experimental.pallas import tpu as pltpu
from jax.experimental.pallas import tpu_sc as plsc

D = 32
P = 20
NNEG = 50
J = P + NNEG
JP = 72
JW = 128
NLANE = 16
NWORKERS = 32
GROUP = 8


def _sc_scores(c2, idx_flat, o_lin):
    B = c2.shape[0]
    nb_per_w = B // NWORKERS
    ngroups = nb_per_w // GROUP
    GJP = GROUP * JP
    NBUF = 4

    mesh = plsc.VectorSubcoreMesh(core_axis_name="c", subcore_axis_name="s")

    @functools.partial(
        pl.kernel,
        out_type=jax.ShapeDtypeStruct((B, JW), jnp.float32),
        mesh=mesh,
        compiler_params=pltpu.CompilerParams(
            needs_layout_passes=False, use_tc_tiling_on_sc=False),
        scratch_types=[
            pltpu.VMEM((NBUF * GJP,), jnp.int32),
            pltpu.VMEM((NBUF * GROUP, D), jnp.float32),
            pltpu.VMEM((NBUF * GJP, D // 2), jnp.int32),
            pltpu.VMEM((NBUF * GROUP, JW), jnp.float32),
            [pltpu.SemaphoreType.DMA for _ in range(NBUF)],
            [pltpu.SemaphoreType.DMA for _ in range(NBUF)],
        ],
    )
    def sc_kernel(c2_hbm, idx_hbm, o_emb_hbm, out_hbm,
                  idxs_v, crows_v, rows_v, scores_v, sems_rows, sems_out):
        ncores = 2
        wid = lax.axis_index("s") * ncores + lax.axis_index("c")
        base_b = wid * nb_per_w

        lanes = lax.broadcasted_iota(jnp.int32, (NLANE,), 0)
        tail_mask = lanes < (J - 4 * NLANE)
        lt = [lanes + t * NLANE for t in range(5)]

        def fetch_group(g, slot):
            b0 = base_b + g * GROUP
            pltpu.sync_copy(idx_hbm.at[pl.ds(b0 * JP, GJP)],
                            idxs_v.at[pl.ds(slot * GJP, GJP)])
            pltpu.async_copy(c2_hbm.at[pl.ds(b0, GROUP), pl.ds(0, D)],
                             crows_v.at[pl.ds(slot * GROUP, GROUP)],
                             sems_rows[slot])
            pltpu.async_copy(o_emb_hbm.at[idxs_v.at[pl.ds(slot * GJP, GJP)]],
                             rows_v.at[pl.ds(slot * GJP, GJP)],
                             sems_rows[slot])

        def drain_group(g, slot):
            b0 = base_b + g * GROUP
            pltpu.make_async_copy(
                c2_hbm.at[pl.ds(b0, GROUP), pl.ds(0, D)],
                crows_v.at[pl.ds(slot * GROUP, GROUP)],
                sems_rows[slot]).wait()
            pltpu.make_async_copy(
                o_emb_hbm.at[idxs_v.at[pl.ds(slot * GJP, GJP)]],
                rows_v.at[pl.ds(slot * GJP, GJP)],
                sems_rows[slot]).wait()

        def compute_group(g, slot):
            drain_group(g, slot)

            def b_body(bi, _):
                accs = [jnp.zeros((NLANE,), jnp.float32) for _ in range(5)]
                row_i = jnp.broadcast_to(bi + slot * GROUP,
                                         (NLANE,)).astype(jnp.int32)
                rbase = row_i * JP
                for w in range(D // 2):
                    wvec = (lanes + w) & (D // 2 - 1)
                    ca = plsc.load_gather(crows_v, [row_i, 2 * wvec])
                    cb = plsc.load_gather(crows_v, [row_i, 2 * wvec + 1])
                    for t in range(5):
                        ridx = rbase + lt[t]
                        if t < 4:
                            colw = plsc.load_gather(rows_v, [ridx, wvec])
                        else:
                            colw = plsc.load_gather(rows_v, [ridx, wvec],
                                                    mask=tail_mask)
                        a, b = plsc.unpack(
                            plsc.bitcast(colw, jnp.bfloat16),
                            format=plsc.PackFormat.INTERLEAVED)
                        accs[t] = accs[t] + a * ca + b * cb
                for t in range(5):
                    scores_v[bi + slot * GROUP,
                             pl.ds(t * NLANE, NLANE)] = accs[t]
                return 0

            lax.fori_loop(0, GROUP, b_body, 0)
            b0 = base_b + g * GROUP
            pltpu.async_copy(
                scores_v.at[pl.ds(slot * GROUP, GROUP)],
                out_hbm.at[pl.ds(b0, GROUP)], sems_out[slot])

        def drain_out(g, slot):
            b0 = base_b + g * GROUP
            pltpu.make_async_copy(
                scores_v.at[pl.ds(slot * GROUP, GROUP)],
                out_hbm.at[pl.ds(b0, GROUP)],
                sems_out[slot]).wait()

        for s in range(NBUF - 1):
            fetch_group(s, s)

        def quad_body(i, _):
            g0 = 4 * i
            for k in range(4):
                g = g0 + k

                @pl.when(g + NBUF - 1 < ngroups)
                def _():
                    fetch_group(g + NBUF - 1, (k + NBUF - 1) % NBUF)

                @pl.when(g >= NBUF)
                def _():
                    drain_out(g - NBUF, k)
                compute_group(g, k)
            return 0

        lax.fori_loop(0, ngroups // 4, quad_body, 0)
        for s in range(NBUF):
            drain_out(ngroups - NBUF + s, s)

    return sc_kernel(c2, idx_flat, o_lin)


def _tc_loss(scores):
    B = scores.shape[0]
    BLK = 512

    def tc_kernel(s_ref, o_ref):
        x = s_ref[...]
        jcol = lax.broadcasted_iota(jnp.int32, x.shape, 1)
        z = jnp.where(jcol < P, x, -x)
        ls = jax.nn.log_sigmoid(z)
        ls = jnp.where(jcol < J, ls, 0.0)
        o_ref[...] = -jnp.sum(ls, axis=1, keepdims=True)

    out = pl.pallas_call(
        tc_kernel,
        out_shape=jax.ShapeDtypeStruct((B, 1), jnp.float32),
        grid=(B // BLK,),
        in_specs=[pl.BlockSpec((BLK, JW), lambda i: (i, 0))],
        out_specs=pl.BlockSpec((BLK, 1), lambda i: (i, 0)),
    )(scores)
    return out.reshape(B)


TCH = 2048


def _relayout_table(t):
    V = t.shape[0]
    ot = t.T

    def tr_kernel(x_ref, o_ref):
        x = x_ref[...]
        eye = (lax.broadcasted_iota(jnp.int32, (D, D), 0) ==
               lax.broadcasted_iota(jnp.int32, (D, D), 1)).astype(jnp.float32)
        z = jax.lax.dot_general(x, eye, (((0,), (0,)), ((), ())),
                                preferred_element_type=jnp.float32)
        zb = z.astype(jnp.bfloat16)
        pieces = []
        for m in range(TCH // 512):
            pieces.append(jnp.concatenate(
                [zb[512 * m + 128 * k: 512 * m + 128 * (k + 1), :]
                 for k in range(4)], axis=1))
        o_ref[...] = jnp.concatenate(pieces, axis=0)

    out2 = pl.pallas_call(
        tr_kernel,
        out_shape=jax.ShapeDtypeStruct((V // 4, 128), jnp.bfloat16),
        grid=(V // TCH,),
        in_specs=[pl.BlockSpec((D, TCH), lambda i: (0, i))],
        out_specs=pl.BlockSpec((TCH // 4, 128), lambda i: (i, 0)),
    )(ot)
    return jax.lax.bitcast_convert_type(
        out2.reshape(V // 4, 64, 2), jnp.int32).reshape(V, D // 2)


def _permute_ids(v):
    v = v.astype(jnp.int32)
    return (v & jnp.int32(-512)) | ((v & 127) << 2) | ((v >> 7) & 3)


@jax.jit
def kernel(c_word, bg_word_pos, bg_word_neg, i_emb, o_emb):
    B = c_word.shape[0]
    idx_flat = jnp.concatenate(
        [_permute_ids(bg_word_pos), _permute_ids(bg_word_neg),
         jnp.zeros((B, JP - J), jnp.int32)], axis=1).reshape(-1)
    c_code = jnp.take(i_emb, c_word, axis=0)
    c2 = jnp.pad(c_code, ((0, 0), (0, JW - D)))
    scores = _sc_scores(c2, idx_flat, _relayout_table(o_emb))
    return _tc_loss(scores)


# speedup vs baseline: 1.8014x; 1.8014x over previous
import functools

import jax
import jax.numpy as jnp
from jax import lax
from jax.experimental import pallas as pl
from jax.experimental.pallas import tpu as pltpu
from jax.experimental.pallas import tpu_sc as plsc

D = 32
P = 20
NNEG = 50
J = P + NNEG
JP = 72
JW = 128
NLANE = 16
NWORKERS = 32
GROUP = 8


def _sc_scores(c2, idx_flat, o_lin):
    B = c2.shape[0]
    nb_per_w = B // NWORKERS
    ngroups = nb_per_w // GROUP
    GJP = GROUP * JP
    NBUF = 4

    mesh = plsc.VectorSubcoreMesh(core_axis_name="c", subcore_axis_name="s")

    @functools.partial(
        pl.kernel,
        out_type=jax.ShapeDtypeStruct((B, JW), jnp.float32),
        mesh=mesh,
        compiler_params=pltpu.CompilerParams(
            needs_layout_passes=False, use_tc_tiling_on_sc=False),
        scratch_types=[
            pltpu.VMEM((NBUF * GJP,), jnp.int32),
            pltpu.VMEM((NBUF * GROUP, D), jnp.float32),
            pltpu.VMEM((NBUF * GJP, D // 2), jnp.int32),
            pltpu.VMEM((NBUF * GROUP, JW), jnp.float32),
            [pltpu.SemaphoreType.DMA for _ in range(NBUF)],
            [pltpu.SemaphoreType.DMA for _ in range(NBUF)],
        ],
    )
    def sc_kernel(c2_hbm, idx_hbm, o_emb_hbm, out_hbm,
                  idxs_v, crows_v, rows_v, scores_v, sems_rows, sems_out):
        ncores = 2
        wid = lax.axis_index("s") * ncores + lax.axis_index("c")
        base_b = wid * nb_per_w

        lanes = lax.broadcasted_iota(jnp.int32, (NLANE,), 0)
        tail_mask = lanes < (J - 4 * NLANE)
        lt = [lanes + t * NLANE for t in range(5)]

        def fetch_group(g, slot):
            b0 = base_b + g * GROUP
            pltpu.sync_copy(idx_hbm.at[pl.ds(b0 * JP, GJP)],
                            idxs_v.at[pl.ds(slot * GJP, GJP)])
            pltpu.async_copy(c2_hbm.at[pl.ds(b0, GROUP), pl.ds(0, D)],
                             crows_v.at[pl.ds(slot * GROUP, GROUP)],
                             sems_rows[slot])
            pltpu.async_copy(o_emb_hbm.at[idxs_v.at[pl.ds(slot * GJP, GJP)]],
                             rows_v.at[pl.ds(slot * GJP, GJP)],
                             sems_rows[slot])

        def drain_group(g, slot):
            b0 = base_b + g * GROUP
            pltpu.make_async_copy(
                c2_hbm.at[pl.ds(b0, GROUP), pl.ds(0, D)],
                crows_v.at[pl.ds(slot * GROUP, GROUP)],
                sems_rows[slot]).wait()
            pltpu.make_async_copy(
                o_emb_hbm.at[idxs_v.at[pl.ds(slot * GJP, GJP)]],
                rows_v.at[pl.ds(slot * GJP, GJP)],
                sems_rows[slot]).wait()

        def compute_group(g, slot):
            drain_group(g, slot)

            def b_body(bi, _):
                accs = [jnp.zeros((NLANE,), jnp.float32) for _ in range(5)]
                row_i = jnp.broadcast_to(bi + slot * GROUP,
                                         (NLANE,)).astype(jnp.int32)
                rbase = row_i * JP
                for w in range(D // 2):
                    wvec = (lanes + w) & (D // 2 - 1)
                    ca = plsc.load_gather(crows_v, [row_i, wvec])
                    cb = plsc.load_gather(crows_v, [row_i, wvec + D // 2])
                    for t in range(5):
                        ridx = rbase + lt[t]
                        if t < 4:
                            colw = plsc.load_gather(rows_v, [ridx, wvec])
                        else:
                            colw = plsc.load_gather(rows_v, [ridx, wvec],
                                                    mask=tail_mask)
                        a, b = plsc.unpack(
                            plsc.bitcast(colw, jnp.bfloat16),
                            format=plsc.PackFormat.INTERLEAVED)
                        accs[t] = accs[t] + a * ca + b * cb
                for t in range(5):
                    scores_v[bi + slot * GROUP,
                             pl.ds(t * NLANE, NLANE)] = accs[t]
                return 0

            lax.fori_loop(0, GROUP, b_body, 0)
            b0 = base_b + g * GROUP
            pltpu.async_copy(
                scores_v.at[pl.ds(slot * GROUP, GROUP)],
                out_hbm.at[pl.ds(b0, GROUP)], sems_out[slot])

        def drain_out(g, slot):
            b0 = base_b + g * GROUP
            pltpu.make_async_copy(
                scores_v.at[pl.ds(slot * GROUP, GROUP)],
                out_hbm.at[pl.ds(b0, GROUP)],
                sems_out[slot]).wait()

        for s in range(NBUF - 1):
            fetch_group(s, s)

        def quad_body(i, _):
            g0 = 4 * i
            for k in range(4):
                g = g0 + k

                @pl.when(g + NBUF - 1 < ngroups)
                def _():
                    fetch_group(g + NBUF - 1, (k + NBUF - 1) % NBUF)

                @pl.when(g >= NBUF)
                def _():
                    drain_out(g - NBUF, k)
                compute_group(g, k)
            return 0

        lax.fori_loop(0, ngroups // 4, quad_body, 0)
        for s in range(NBUF):
            drain_out(ngroups - NBUF + s, s)

    return sc_kernel(c2, idx_flat, o_lin)


def _tc_loss(scores):
    B = scores.shape[0]
    BLK = 512

    def tc_kernel(s_ref, o_ref):
        x = s_ref[...]
        jcol = lax.broadcasted_iota(jnp.int32, x.shape, 1)
        z = jnp.where(jcol < P, x, -x)
        ls = jax.nn.log_sigmoid(z)
        ls = jnp.where(jcol < J, ls, 0.0)
        o_ref[...] = -jnp.sum(ls, axis=1, keepdims=True)

    out = pl.pallas_call(
        tc_kernel,
        out_shape=jax.ShapeDtypeStruct((B, 1), jnp.float32),
        grid=(B // BLK,),
        in_specs=[pl.BlockSpec((BLK, JW), lambda i: (i, 0))],
        out_specs=pl.BlockSpec((BLK, 1), lambda i: (i, 0)),
    )(scores)
    return out.reshape(B)


TCH = 2048


def _relayout_table(t):
    V = t.shape[0]
    ot = t.T

    def tr_kernel(x_ref, o_ref):
        x = x_ref[...]
        eye = (lax.broadcasted_iota(jnp.int32, (D, D), 0) ==
               lax.broadcasted_iota(jnp.int32, (D, D), 1)).astype(jnp.float32)
        z = jax.lax.dot_general(x, eye, (((0,), (0,)), ((), ())),
                                preferred_element_type=jnp.float32)
        lo = jax.lax.bitcast_convert_type(
            z[:, :D // 2].astype(jnp.bfloat16), jnp.uint16)
        hi = jax.lax.bitcast_convert_type(
            z[:, D // 2:].astype(jnp.bfloat16), jnp.uint16)
        zi = (lo.astype(jnp.uint32)
              | (hi.astype(jnp.uint32) << 16)).astype(jnp.int32)
        pieces = []
        for m in range(TCH // 1024):
            pieces.append(jnp.concatenate(
                [zi[1024 * m + 128 * k: 1024 * m + 128 * (k + 1), :]
                 for k in range(8)], axis=1))
        o_ref[...] = jnp.concatenate(pieces, axis=0)

    out3 = pl.pallas_call(
        tr_kernel,
        out_shape=jax.ShapeDtypeStruct((V // 8, 128), jnp.int32),
        grid=(V // TCH,),
        in_specs=[pl.BlockSpec((D, TCH), lambda i: (0, i))],
        out_specs=pl.BlockSpec((TCH // 8, 128), lambda i: (i, 0)),
    )(ot)
    return out3.reshape(V, D // 2)


def _permute_ids(v):
    v = v.astype(jnp.int32)
    return (v & jnp.int32(-1024)) | ((v & 127) << 3) | ((v >> 7) & 7)


@jax.jit
def kernel(c_word, bg_word_pos, bg_word_neg, i_emb, o_emb):
    B = c_word.shape[0]
    idx_flat = jnp.concatenate(
        [_permute_ids(bg_word_pos), _permute_ids(bg_word_neg),
         jnp.zeros((B, JP - J), jnp.int32)], axis=1).reshape(-1)
    c_code = jnp.take(i_emb, c_word, axis=0)
    c2 = jnp.pad(c_code, ((0, 0), (0, JW - D)))
    scores = _sc_scores(c2, idx_flat, _relayout_table(o_emb))
    return _tc_loss(scores)


# speedup vs baseline: 2.1533x; 1.1954x over previous
import functools

import jax
import jax.numpy as jnp
from jax import lax
from jax.experimental import pallas as pl
from jax.experimental.pallas import tpu as pltpu
from jax.experimental.pallas import tpu_sc as plsc

D = 32
P = 20
NNEG = 50
J = P + NNEG
JP = 72
JW = 128
NLANE = 16
NWORKERS = 32
GROUP = 8


def _sc_scores(c2, idx_flat, o_lin):
    B = c2.shape[0]
    nb_per_w = B // NWORKERS
    ngroups = nb_per_w // GROUP
    GJP = GROUP * JP
    NBUF = 4

    mesh = plsc.VectorSubcoreMesh(core_axis_name="c", subcore_axis_name="s")

    @functools.partial(
        pl.kernel,
        out_type=jax.ShapeDtypeStruct((B, JW), jnp.float32),
        mesh=mesh,
        compiler_params=pltpu.CompilerParams(
            needs_layout_passes=False, use_tc_tiling_on_sc=False),
        scratch_types=[
            pltpu.VMEM((NBUF * GJP,), jnp.int32),
            pltpu.VMEM((NBUF * GROUP, D), jnp.float32),
            pltpu.VMEM((NBUF * GJP, D // 2), jnp.int32),
            pltpu.VMEM((NBUF * GROUP, JW), jnp.float32),
            [pltpu.SemaphoreType.DMA for _ in range(NBUF)],
            [pltpu.SemaphoreType.DMA for _ in range(NBUF)],
        ],
    )
    def sc_kernel(c2_hbm, idx_hbm, o_emb_hbm, out_hbm,
                  idxs_v, crows_v, rows_v, scores_v, sems_rows, sems_out):
        ncores = 2
        wid = lax.axis_index("s") * ncores + lax.axis_index("c")
        base_b = wid * nb_per_w

        lanes = lax.broadcasted_iota(jnp.int32, (NLANE,), 0)
        tail_mask = lanes < (J - 4 * NLANE)
        lt = [lanes + t * NLANE for t in range(5)]

        def fetch_group(g, slot):
            b0 = base_b + g * GROUP
            pltpu.sync_copy(idx_hbm.at[pl.ds(b0 * JP, GJP)],
                            idxs_v.at[pl.ds(slot * GJP, GJP)])
            pltpu.async_copy(c2_hbm.at[pl.ds(b0, GROUP), pl.ds(0, D)],
                             crows_v.at[pl.ds(slot * GROUP, GROUP)],
                             sems_rows[slot])
            pltpu.async_copy(o_emb_hbm.at[idxs_v.at[pl.ds(slot * GJP, GJP)]],
                             rows_v.at[pl.ds(slot * GJP, GJP)],
                             sems_rows[slot])

        def drain_group(g, slot):
            b0 = base_b + g * GROUP
            pltpu.make_async_copy(
                c2_hbm.at[pl.ds(b0, GROUP), pl.ds(0, D)],
                crows_v.at[pl.ds(slot * GROUP, GROUP)],
                sems_rows[slot]).wait()
            pltpu.make_async_copy(
                o_emb_hbm.at[idxs_v.at[pl.ds(slot * GJP, GJP)]],
                rows_v.at[pl.ds(slot * GJP, GJP)],
                sems_rows[slot]).wait()

        def compute_group(g, slot):
            drain_group(g, slot)

            def b_body(bi, _):
                accs = [jnp.zeros((NLANE,), jnp.float32) for _ in range(5)]
                row_i = jnp.broadcast_to(bi + slot * GROUP,
                                         (NLANE,)).astype(jnp.int32)
                rbase = row_i * JP
                for w in range(D // 2):
                    wvec = (lanes + w) & (D // 2 - 1)
                    ca = plsc.load_gather(crows_v, [row_i, wvec])
                    cb = plsc.load_gather(crows_v, [row_i, wvec + D // 2])
                    for t in range(5):
                        ridx = rbase + lt[t]
                        if t < 4:
                            colw = plsc.load_gather(rows_v, [ridx, wvec])
                        else:
                            colw = plsc.load_gather(rows_v, [ridx, wvec],
                                                    mask=tail_mask)
                        a, b = plsc.unpack(
                            plsc.bitcast(colw, jnp.bfloat16),
                            format=plsc.PackFormat.INTERLEAVED)
                        accs[t] = accs[t] + a * ca + b * cb
                for t in range(5):
                    scores_v[bi + slot * GROUP,
                             pl.ds(t * NLANE, NLANE)] = accs[t]
                return 0

            lax.fori_loop(0, GROUP, b_body, 0)
            b0 = base_b + g * GROUP
            pltpu.async_copy(
                scores_v.at[pl.ds(slot * GROUP, GROUP)],
                out_hbm.at[pl.ds(b0, GROUP)], sems_out[slot])

        def drain_out(g, slot):
            b0 = base_b + g * GROUP
            pltpu.make_async_copy(
                scores_v.at[pl.ds(slot * GROUP, GROUP)],
                out_hbm.at[pl.ds(b0, GROUP)],
                sems_out[slot]).wait()

        for s in range(NBUF - 1):
            fetch_group(s, s)

        def quad_body(i, _):
            g0 = 4 * i
            for k in range(4):
                g = g0 + k

                @pl.when(g + NBUF - 1 < ngroups)
                def _():
                    fetch_group(g + NBUF - 1, (k + NBUF - 1) % NBUF)

                @pl.when(g >= NBUF)
                def _():
                    drain_out(g - NBUF, k)
                compute_group(g, k)
            return 0

        lax.fori_loop(0, ngroups // 4, quad_body, 0)
        for s in range(NBUF):
            drain_out(ngroups - NBUF + s, s)

    return sc_kernel(c2, idx_flat, o_lin)


def _tc_loss(scores):
    B = scores.shape[0]
    BLK = 512

    def tc_kernel(s_ref, o_ref):
        x = s_ref[...]
        jcol = lax.broadcasted_iota(jnp.int32, x.shape, 1)
        z = jnp.where(jcol < P, x, -x)
        ls = jax.nn.log_sigmoid(z)
        ls = jnp.where(jcol < J, ls, 0.0)
        o_ref[...] = -jnp.sum(ls, axis=1, keepdims=True)

    out = pl.pallas_call(
        tc_kernel,
        out_shape=jax.ShapeDtypeStruct((B, 1), jnp.float32),
        grid=(B // BLK,),
        in_specs=[pl.BlockSpec((BLK, JW), lambda i: (i, 0))],
        out_specs=pl.BlockSpec((BLK, 1), lambda i: (i, 0)),
    )(scores)
    return out.reshape(B)


TCH = 8192


def _relayout_table(t):
    V = t.shape[0]
    ot = t.T

    def tr_kernel(x_ref, o_ref):
        x = x_ref[...]
        eye = (lax.broadcasted_iota(jnp.int32, (D, D), 0) ==
               lax.broadcasted_iota(jnp.int32, (D, D), 1)).astype(jnp.float32)
        z = jax.lax.dot_general(x, eye, (((0,), (0,)), ((), ())),
                                preferred_element_type=jnp.float32)
        lo = jax.lax.bitcast_convert_type(
            z[:, :D // 2].astype(jnp.bfloat16), jnp.uint16)
        hi = jax.lax.bitcast_convert_type(
            z[:, D // 2:].astype(jnp.bfloat16), jnp.uint16)
        zi = (lo.astype(jnp.uint32)
              | (hi.astype(jnp.uint32) << 16)).astype(jnp.int32)
        pieces = []
        for m in range(TCH // 1024):
            pieces.append(jnp.concatenate(
                [zi[1024 * m + 128 * k: 1024 * m + 128 * (k + 1), :]
                 for k in range(8)], axis=1))
        o_ref[...] = jnp.concatenate(pieces, axis=0)

    out3 = pl.pallas_call(
        tr_kernel,
        out_shape=jax.ShapeDtypeStruct((V // 8, 128), jnp.int32),
        grid=(V // TCH,),
        in_specs=[pl.BlockSpec((D, TCH), lambda i: (0, i))],
        out_specs=pl.BlockSpec((TCH // 8, 128), lambda i: (i, 0)),
    )(ot)
    return out3.reshape(V, D // 2)


def _permute_ids(v):
    v = v.astype(jnp.int32)
    return (v & jnp.int32(-1024)) | ((v & 127) << 3) | ((v >> 7) & 7)


@jax.jit
def kernel(c_word, bg_word_pos, bg_word_neg, i_emb, o_emb):
    B = c_word.shape[0]
    idx_flat = jnp.concatenate(
        [_permute_ids(bg_word_pos), _permute_ids(bg_word_neg),
         jnp.zeros((B, JP - J), jnp.int32)], axis=1).reshape(-1)
    c_code = jnp.take(i_emb, c_word, axis=0)
    c2 = jnp.pad(c_code, ((0, 0), (0, JW - D)))
    scores = _sc_scores(c2, idx_flat, _relayout_table(o_emb))
    return _tc_loss(scores)


# speedup vs baseline: 2.1757x; 1.0104x over previous
import functools

import jax
import jax.numpy as jnp
from jax import lax
from jax.experimental import pallas as pl
from jax.experimental.pallas import tpu as pltpu
from jax.experimental.pallas import tpu_sc as plsc

D = 32
P = 20
NNEG = 50
J = P + NNEG
JP = 72
JW = 128
NLANE = 16
NWORKERS = 32
GROUP = 8


def _sc_scores(c2, idx_flat, o_lin):
    B = c2.shape[0]
    nb_per_w = B // NWORKERS
    ngroups = nb_per_w // GROUP
    GJP = GROUP * JP
    NBUF = 4

    mesh = plsc.VectorSubcoreMesh(core_axis_name="c", subcore_axis_name="s")

    @functools.partial(
        pl.kernel,
        out_type=jax.ShapeDtypeStruct((B, JW), jnp.float32),
        mesh=mesh,
        compiler_params=pltpu.CompilerParams(
            needs_layout_passes=False, use_tc_tiling_on_sc=False),
        scratch_types=[
            pltpu.VMEM((NBUF * GJP,), jnp.int32),
            pltpu.VMEM((NBUF * GROUP, D), jnp.float32),
            pltpu.VMEM((NBUF * GJP, D // 2), jnp.int32),
            pltpu.VMEM((NBUF * GROUP, JW), jnp.float32),
            [pltpu.SemaphoreType.DMA for _ in range(NBUF)],
            [pltpu.SemaphoreType.DMA for _ in range(NBUF)],
        ],
    )
    def sc_kernel(c2_hbm, idx_hbm, o_emb_hbm, out_hbm,
                  idxs_v, crows_v, rows_v, scores_v, sems_rows, sems_out):
        ncores = 2
        wid = lax.axis_index("s") * ncores + lax.axis_index("c")
        base_b = wid * nb_per_w

        lanes = lax.broadcasted_iota(jnp.int32, (NLANE,), 0)
        tail_mask = lanes < (J - 4 * NLANE)
        lt = [lanes + t * NLANE for t in range(5)]

        def fetch_group(g, slot):
            b0 = base_b + g * GROUP
            pltpu.sync_copy(idx_hbm.at[pl.ds(b0 * JP, GJP)],
                            idxs_v.at[pl.ds(slot * GJP, GJP)])
            pltpu.async_copy(c2_hbm.at[pl.ds(b0, GROUP), pl.ds(0, D)],
                             crows_v.at[pl.ds(slot * GROUP, GROUP)],
                             sems_rows[slot])
            pltpu.async_copy(o_emb_hbm.at[idxs_v.at[pl.ds(slot * GJP, GJP)]],
                             rows_v.at[pl.ds(slot * GJP, GJP)],
                             sems_rows[slot])

        def drain_group(g, slot):
            b0 = base_b + g * GROUP
            pltpu.make_async_copy(
                c2_hbm.at[pl.ds(b0, GROUP), pl.ds(0, D)],
                crows_v.at[pl.ds(slot * GROUP, GROUP)],
                sems_rows[slot]).wait()
            pltpu.make_async_copy(
                o_emb_hbm.at[idxs_v.at[pl.ds(slot * GJP, GJP)]],
                rows_v.at[pl.ds(slot * GJP, GJP)],
                sems_rows[slot]).wait()

        def compute_group(g, slot):
            drain_group(g, slot)

            def b_body(bi, _):
                accs = [jnp.zeros((NLANE,), jnp.float32) for _ in range(5)]
                row_i = jnp.broadcast_to(bi + slot * GROUP,
                                         (NLANE,)).astype(jnp.int32)
                rbase = row_i * JP
                for w in range(D // 2):
                    wvec = (lanes + w) & (D // 2 - 1)
                    ca = plsc.load_gather(crows_v, [row_i, wvec])
                    cb = plsc.load_gather(crows_v, [row_i, wvec + D // 2])
                    for t in range(5):
                        ridx = rbase + lt[t]
                        if t < 4:
                            colw = plsc.load_gather(rows_v, [ridx, wvec])
                        else:
                            colw = plsc.load_gather(rows_v, [ridx, wvec],
                                                    mask=tail_mask)
                        a, b = plsc.unpack(
                            plsc.bitcast(colw, jnp.bfloat16),
                            format=plsc.PackFormat.INTERLEAVED)
                        accs[t] = accs[t] + a * ca + b * cb
                for t in range(5):
                    scores_v[bi + slot * GROUP,
                             pl.ds(t * NLANE, NLANE)] = accs[t]
                return 0

            lax.fori_loop(0, GROUP, b_body, 0)
            b0 = base_b + g * GROUP
            pltpu.async_copy(
                scores_v.at[pl.ds(slot * GROUP, GROUP)],
                out_hbm.at[pl.ds(b0, GROUP)], sems_out[slot])

        def drain_out(g, slot):
            b0 = base_b + g * GROUP
            pltpu.make_async_copy(
                scores_v.at[pl.ds(slot * GROUP, GROUP)],
                out_hbm.at[pl.ds(b0, GROUP)],
                sems_out[slot]).wait()

        for s in range(NBUF - 1):
            fetch_group(s, s)

        def quad_body(i, _):
            g0 = 4 * i
            for k in range(4):
                g = g0 + k

                @pl.when(g + NBUF - 1 < ngroups)
                def _():
                    fetch_group(g + NBUF - 1, (k + NBUF - 1) % NBUF)

                @pl.when(g >= NBUF)
                def _():
                    drain_out(g - NBUF, k)
                compute_group(g, k)
            return 0

        lax.fori_loop(0, ngroups // 4, quad_body, 0)
        for s in range(NBUF):
            drain_out(ngroups - NBUF + s, s)

    return sc_kernel(c2, idx_flat, o_lin)


def _tc_loss(scores):
    B = scores.shape[0]
    BLK = 512

    def tc_kernel(s_ref, o_ref):
        x = s_ref[...]
        jcol = lax.broadcasted_iota(jnp.int32, x.shape, 1)
        z = jnp.where(jcol < P, x, -x)
        ls = jax.nn.log_sigmoid(z)
        ls = jnp.where(jcol < J, ls, 0.0)
        o_ref[...] = -jnp.sum(ls, axis=1, keepdims=True)

    out = pl.pallas_call(
        tc_kernel,
        out_shape=jax.ShapeDtypeStruct((B, 1), jnp.float32),
        grid=(B // BLK,),
        in_specs=[pl.BlockSpec((BLK, JW), lambda i: (i, 0))],
        out_specs=pl.BlockSpec((BLK, 1), lambda i: (i, 0)),
    )(scores)
    return out.reshape(B)


TCH = 16384


def _relayout_table(t):
    V = t.shape[0]
    ot = t.T

    def tr_kernel(x_ref, o_ref):
        x = x_ref[...]
        eye = (lax.broadcasted_iota(jnp.int32, (D, D), 0) ==
               lax.broadcasted_iota(jnp.int32, (D, D), 1)).astype(jnp.float32)
        z = jax.lax.dot_general(x, eye, (((0,), (0,)), ((), ())),
                                preferred_element_type=jnp.float32)
        lo = jax.lax.bitcast_convert_type(
            z[:, :D // 2].astype(jnp.bfloat16), jnp.uint16)
        hi = jax.lax.bitcast_convert_type(
            z[:, D // 2:].astype(jnp.bfloat16), jnp.uint16)
        zi = (lo.astype(jnp.uint32)
              | (hi.astype(jnp.uint32) << 16)).astype(jnp.int32)
        pieces = []
        for m in range(TCH // 1024):
            pieces.append(jnp.concatenate(
                [zi[1024 * m + 128 * k: 1024 * m + 128 * (k + 1), :]
                 for k in range(8)], axis=1))
        o_ref[...] = jnp.concatenate(pieces, axis=0)

    out3 = pl.pallas_call(
        tr_kernel,
        out_shape=jax.ShapeDtypeStruct((V // 8, 128), jnp.int32),
        grid=(V // TCH,),
        in_specs=[pl.BlockSpec((D, TCH), lambda i: (0, i))],
        out_specs=pl.BlockSpec((TCH // 8, 128), lambda i: (i, 0)),
    )(ot)
    return out3.reshape(V, D // 2)


def _permute_ids(v):
    v = v.astype(jnp.int32)
    return (v & jnp.int32(-1024)) | ((v & 127) << 3) | ((v >> 7) & 7)


@jax.jit
def kernel(c_word, bg_word_pos, bg_word_neg, i_emb, o_emb):
    B = c_word.shape[0]
    idx_flat = jnp.concatenate(
        [_permute_ids(bg_word_pos), _permute_ids(bg_word_neg),
         jnp.zeros((B, JP - J), jnp.int32)], axis=1).reshape(-1)
    c_code = jnp.take(i_emb, c_word, axis=0)
    c2 = jnp.pad(c_code, ((0, 0), (0, JW - D)))
    scores = _sc_scores(c2, idx_flat, _relayout_table(o_emb))
    return _tc_loss(scores)
